# baseline (speedup 1.0000x reference)
import jax
import jax.numpy as jnp
from jax import lax
from jax.experimental import pallas as pl
from jax.experimental.pallas import tpu as pltpu
from jax.experimental.pallas import tpu_sc as plsc

_HID = 1024
_INTER = 4096
_E = 64
_NR = 32
_NF = 32
_T = 2048

_CB = 256
_NB_BASE = _INTER // _CB
_NB_AD = (_E * _NR) // _CB
_EPB = _CB // _NR

_TT = 256
_NTT = _T // _TT

_NC = 2
_NS = 16


def _sc_sort_body(idx_hbm, scales_hbm, x_hbm,
                  inv_hbm, pos_hbm, idxs_hbm, sr_hbm, sf_hbm, xs_hbm,
                  idx_v, tok_v, pos_v, post_v, idxs_v, sr_v, sf_v, scales_v,
                  allh_v, cnt_v, off_v, buf_v, buf2_v, myinv_v, rows_v,
                  sh_hist, sh_inv, sh_idxs, sh_sr, sh_sf,
                  sem):
    core = lax.axis_index("c")
    sid = lax.axis_index("s")
    base = sid * 128
    l16 = lax.iota(jnp.int32, 16)

    pltpu.sync_copy(idx_hbm.at[pl.ds(base, 128)], idx_v)
    pltpu.sync_copy(scales_hbm, scales_v)

    def runs(c):
        k16 = idx_v[pl.ds(16 * c, 16)]
        ks, vs = plsc.sort_key_val(k16, l16)
        buf_v[pl.ds(0, 16)] = jnp.full((16,), -1, jnp.int32)
        buf_v[pl.ds(1, 16)] = ks
        prev = buf_v[pl.ds(0, 16)]
        buf2_v[pl.ds(1, 16)] = jnp.full((16,), -2, jnp.int32)
        buf2_v[pl.ds(0, 16)] = ks
        nxt = buf2_v[pl.ds(1, 16)]
        is_new = ks != prev
        last = ks != nxt
        run_start = plsc.cummax(l16, mask=is_new)
        rank = l16 - run_start
        return ks, vs, rank, last

    for k in range(_E // 16):
        cnt_v[pl.ds(16 * k, 16)] = jnp.zeros((16,), jnp.int32)
    for c in range(8):
        ks, vs, rank, last = runs(c)
        cur = plsc.load_gather(cnt_v, [ks])
        plsc.store_scatter(cnt_v, [ks], cur + rank + 1, mask=last)

    pltpu.sync_copy(cnt_v, sh_hist.at[sid])
    plsc.subcore_barrier()
    pltpu.sync_copy(sh_hist, allh_v)

    sid16 = jnp.full((16,), 0, jnp.int32) + sid
    carry = jnp.int32(0)
    for k in range(_E // 16):
        tot16 = jnp.zeros((16,), jnp.int32)
        mine16 = jnp.zeros((16,), jnp.int32)
        for w in range(_NS):
            row = allh_v[w, pl.ds(16 * k, 16)]
            tot16 = tot16 + row
            wlt = jnp.full((16,), w, jnp.int32) < sid16
            mine16 = mine16 + jnp.where(wlt, row, 0)
        gbase16 = carry + plsc.cumsum(tot16) - tot16
        off_v[pl.ds(16 * k, 16)] = gbase16 + mine16
        carry = carry + jnp.sum(tot16)

    for c in range(8):
        ks, vs, rank, last = runs(c)
        offs = plsc.load_gather(off_v, [ks])
        pos16 = offs + rank
        plsc.store_scatter(off_v, [ks], pos16 + 1, mask=last)
        pos_v[pl.ds(16 * c, 16)] = pos16
        tok_v[pl.ds(16 * c, 16)] = base + 16 * c + vs
        idxs_v[pl.ds(16 * c, 16)] = ks
        sr_v[pl.ds(16 * c, 16)] = plsc.load_gather(scales_v, [2 * ks])
        sf_v[pl.ds(16 * c, 16)] = plsc.load_gather(scales_v, [2 * ks + 1])
        plsc.store_scatter(post_v, [16 * c + vs], pos16)

    pltpu.async_copy(tok_v, sh_inv.at[pos_v], sem).wait()
    pltpu.async_copy(idxs_v, sh_idxs.at[pos_v], sem).wait()
    pltpu.async_copy(sr_v, sh_sr.at[pos_v], sem).wait()
    pltpu.async_copy(sf_v, sh_sf.at[pos_v], sem).wait()
    plsc.subcore_barrier()

    @pl.when(core == 0)
    def _():
        pltpu.sync_copy(sh_inv.at[pl.ds(base, 128)],
                        inv_hbm.at[pl.ds(base, 128)])
        pltpu.sync_copy(sh_idxs.at[pl.ds(base, 128)],
                        idxs_hbm.at[pl.ds(base, 128)])
        pltpu.sync_copy(sh_sr.at[pl.ds(base, 128)],
                        sr_hbm.at[pl.ds(base, 128)])
        pltpu.sync_copy(sh_sf.at[pl.ds(base, 128)],
                        sf_hbm.at[pl.ds(base, 128)])
        pltpu.sync_copy(post_v, pos_hbm.at[pl.ds(base, 128)])

    row0 = core * 1024 + sid * 64
    pltpu.sync_copy(sh_inv.at[pl.ds(row0, 64)], myinv_v)
    for j in range(4):
        pltpu.async_copy(x_hbm.at[myinv_v.at[pl.ds(16 * j, 16)]],
                         rows_v, sem).wait()
        pltpu.sync_copy(rows_v, xs_hbm.at[pl.ds(row0 + 16 * j, 16)])


def _sc_sort(idx, scales_flat, x):
    mesh = plsc.VectorSubcoreMesh(core_axis_name="c", subcore_axis_name="s")
    f = pl.kernel(
        _sc_sort_body,
        mesh=mesh,
        out_type=(
            jax.ShapeDtypeStruct((_T,), jnp.int32),
            jax.ShapeDtypeStruct((_T,), jnp.int32),
            jax.ShapeDtypeStruct((_T,), jnp.int32),
            jax.ShapeDtypeStruct((_T,), jnp.float32),
            jax.ShapeDtypeStruct((_T,), jnp.float32),
            jax.ShapeDtypeStruct((_T, _HID), jnp.float32),
        ),
        scratch_types=[
            pltpu.VMEM((128,), jnp.int32),
            pltpu.VMEM((128,), jnp.int32),
            pltpu.VMEM((128,), jnp.int32),
            pltpu.VMEM((128,), jnp.int32),
            pltpu.VMEM((128,), jnp.int32),
            pltpu.VMEM((128,), jnp.float32),
            pltpu.VMEM((128,), jnp.float32),
            pltpu.VMEM((128,), jnp.float32),
            pltpu.VMEM((_NS, _E), jnp.int32),
            pltpu.VMEM((_E,), jnp.int32),
            pltpu.VMEM((_E,), jnp.int32),
            pltpu.VMEM((32,), jnp.int32),
            pltpu.VMEM((32,), jnp.int32),
            pltpu.VMEM((64,), jnp.int32),
            pltpu.VMEM((16, _HID), jnp.float32),
            pltpu.VMEM_SHARED((_NS, _E), jnp.int32),
            pltpu.VMEM_SHARED((_T,), jnp.int32),
            pltpu.VMEM_SHARED((_T,), jnp.int32),
            pltpu.VMEM_SHARED((_T,), jnp.float32),
            pltpu.VMEM_SHARED((_T,), jnp.float32),
            pltpu.SemaphoreType.DMA,
        ],
        compiler_params=pltpu.CompilerParams(needs_layout_passes=False),
    )
    return f(idx, scales_flat, x)


def _sc_unsort_add_body(pos_hbm, ads_hbm, base_hbm, out_hbm,
                        pos_v, rows_v, base_v, sem):
    core = lax.axis_index("c")
    sid = lax.axis_index("s")
    row0 = core * 1024 + sid * 64
    pltpu.sync_copy(pos_hbm.at[pl.ds(row0, 64)], pos_v)
    for j in range(4):
        cp = pltpu.async_copy(
            ads_hbm.at[pos_v.at[pl.ds(16 * j, 16)]], rows_v, sem)
        pltpu.sync_copy(base_hbm.at[pl.ds(row0 + 16 * j, 16)], base_v)
        cp.wait()

        def addk(k, carry):
            s = pl.ds(k * 16, 16)
            for r in range(16):
                rows_v[r, s] = rows_v[r, s] + base_v[r, s]
            return carry

        lax.fori_loop(0, _HID // 16, addk, jnp.int32(0))
        pltpu.sync_copy(rows_v, out_hbm.at[pl.ds(row0 + 16 * j, 16)])


def _sc_unsort_add(pos, ad_sorted, base_out):
    mesh = plsc.VectorSubcoreMesh(core_axis_name="c", subcore_axis_name="s")
    f = pl.kernel(
        _sc_unsort_add_body,
        mesh=mesh,
        out_type=jax.ShapeDtypeStruct((_T, _HID), jnp.float32),
        scratch_types=[
            pltpu.VMEM((64,), jnp.int32),
            pltpu.VMEM((16, _HID), jnp.float32),
            pltpu.VMEM((16, _HID), jnp.float32),
            pltpu.SemaphoreType.DMA,
        ],
    )
    return f(pos, ad_sorted, base_out)


def _base_body(x_ref, bg_ref, bu_ref, bd_ref, out_ref, xb_s, wg_s, wu_s, bd_s):
    c = pl.program_id(0)
    t = pl.program_id(1)

    @pl.when((c == 0) & (t == 0))
    def _():
        xb_s[...] = x_ref[...].astype(jnp.bfloat16)

    @pl.when(t == 0)
    def _():
        wg_s[...] = bg_ref[...].astype(jnp.bfloat16)
        wu_s[...] = bu_ref[...].astype(jnp.bfloat16)
        bd_s[...] = bd_ref[...].astype(jnp.bfloat16)

    rows = pl.ds(t * _TT, _TT)
    x = xb_s[rows, :]
    dn = (((1,), (1,)), ((), ()))
    g = lax.dot_general(x, wg_s[...], dn, preferred_element_type=jnp.float32)
    u = lax.dot_general(x, wu_s[...], dn, preferred_element_type=jnp.float32)
    sig = 1.0 / (1.0 + jnp.exp(-g))
    h = (g * sig) * u
    contrib = lax.dot_general(h.astype(jnp.bfloat16), bd_s[...], dn,
                              preferred_element_type=jnp.float32)

    @pl.when(c == 0)
    def _():
        out_ref[rows, :] = contrib

    @pl.when(c > 0)
    def _():
        out_ref[rows, :] += contrib


def _tc_base(x, base_gate_w, base_up_w, base_down_w):
    return pl.pallas_call(
        _base_body,
        grid=(_NB_BASE, _NTT),
        in_specs=[
            pl.BlockSpec((_T, _HID), lambda c, t: (0, 0)),
            pl.BlockSpec((_CB, _HID), lambda c, t: (c, 0)),
            pl.BlockSpec((_CB, _HID), lambda c, t: (c, 0)),
            pl.BlockSpec((_HID, _CB), lambda c, t: (0, c)),
        ],
        out_specs=pl.BlockSpec((_T, _HID), lambda c, t: (0, 0)),
        out_shape=jax.ShapeDtypeStruct((_T, _HID), jnp.float32),
        scratch_shapes=[
            pltpu.VMEM((_T, _HID), jnp.bfloat16),
            pltpu.VMEM((_CB, _HID), jnp.bfloat16),
            pltpu.VMEM((_CB, _HID), jnp.bfloat16),
            pltpu.VMEM((_HID, _CB), jnp.bfloat16),
        ],
        compiler_params=pltpu.CompilerParams(
            dimension_semantics=("arbitrary", "arbitrary")),
    )(x, base_gate_w, base_up_w, base_down_w)


def _ad_body(idx_ref, sr_ref, sf_ref, xs_ref, rg_ref, ru_ref, fg_ref, fu_ref,
             rd_ref, fd_ref, out_ref, xb_s, wg_s, wu_s, wd_s):
    c = pl.program_id(0)
    t = pl.program_id(1)
    rows = pl.ds(t * _TT, _TT)

    @pl.when((c == 0) & (t == 0))
    def _():
        xb_s[...] = xs_ref[...].astype(jnp.bfloat16)

    @pl.when(t == 0)
    def _():
        @pl.when(c < _NB_AD)
        def _():
            wg_s[...] = rg_ref[...].astype(jnp.bfloat16)
            wu_s[...] = ru_ref[...].astype(jnp.bfloat16)
            wd_s[...] = rd_ref[...].astype(jnp.bfloat16)

        @pl.when(c >= _NB_AD)
        def _():
            wg_s[...] = fg_ref[...].astype(jnp.bfloat16)
            wu_s[...] = fu_ref[...].astype(jnp.bfloat16)
            wd_s[...] = fd_ref[...].astype(jnp.bfloat16)

    @pl.when(c == 0)
    def _():
        out_ref[rows, :] = jnp.zeros((_TT, _HID), jnp.float32)

    blk = jnp.where(c < _NB_AD, c, c - _NB_AD)
    e0 = blk * _EPB
    idxv = idx_ref[rows, :]
    emin = jnp.min(idxv)
    emax = jnp.max(idxv)

    def run(s_ref):
        x = xb_s[rows, :]
        dn = (((1,), (1,)), ((), ()))
        g = lax.dot_general(x, wg_s[...], dn,
                            preferred_element_type=jnp.float32)
        u = lax.dot_general(x, wu_s[...], dn,
                            preferred_element_type=jnp.float32)
        sig = 1.0 / (1.0 + jnp.exp(-g))
        h = (g * sig) * u
        ecol = e0 + lax.broadcasted_iota(jnp.int32, (_TT, _CB), 1) // _NR
        h = jnp.where(ecol == idxv, h * s_ref[rows, :], 0.0)
        hb = h.astype(jnp.bfloat16)
        contrib = jnp.zeros((_TT, _HID), jnp.float32)
        for e in range(_EPB):
            he = hb[:, e * _NR:(e + 1) * _NR]
            contrib += lax.dot_general(
                he, wd_s[e], (((1,), (1,)), ((), ())),
                preferred_element_type=jnp.float32)
        out_ref[rows, :] += contrib

    @pl.when((emax >= e0) & (emin < e0 + _EPB) & (c < _NB_AD))
    def _():
        run(sr_ref)

    @pl.when((emax >= e0) & (emin < e0 + _EPB) & (c >= _NB_AD))
    def _():
        run(sf_ref)


def _tc_adapters(idxs, sr, sf, x_sorted, rg, ru, fg, fu, rdv, fdv):
    nb = _NB_AD

    def rblk(c, t):
        return (jnp.clip(c, 0, nb - 1), 0)

    def fblk(c, t):
        return (jnp.clip(c - nb, 0, nb - 1), 0)

    def rblk3(c, t):
        return (jnp.clip(c, 0, nb - 1), 0, 0)

    def fblk3(c, t):
        return (jnp.clip(c - nb, 0, nb - 1), 0, 0)

    return pl.pallas_call(
        _ad_body,
        grid=(2 * nb, _NTT),
        in_specs=[
            pl.BlockSpec((_T, 1), lambda c, t: (0, 0)),
            pl.BlockSpec((_T, 1), lambda c, t: (0, 0)),
            pl.BlockSpec((_T, 1), lambda c, t: (0, 0)),
            pl.BlockSpec((_T, _HID), lambda c, t: (0, 0)),
            pl.BlockSpec((_CB, _HID), rblk),
            pl.BlockSpec((_CB, _HID), rblk),
            pl.BlockSpec((_CB, _HID), fblk),
            pl.BlockSpec((_CB, _HID), fblk),
            pl.BlockSpec((_EPB, _HID, _NR), rblk3),
            pl.BlockSpec((_EPB, _HID, _NF), fblk3),
        ],
        out_specs=pl.BlockSpec((_T, _HID), lambda c, t: (0, 0)),
        out_shape=jax.ShapeDtypeStruct((_T, _HID), jnp.float32),
        scratch_shapes=[
            pltpu.VMEM((_T, _HID), jnp.bfloat16),
            pltpu.VMEM((_CB, _HID), jnp.bfloat16),
            pltpu.VMEM((_CB, _HID), jnp.bfloat16),
            pltpu.VMEM((_EPB, _HID, _NR), jnp.bfloat16),
        ],
        compiler_params=pltpu.CompilerParams(
            dimension_semantics=("arbitrary", "arbitrary")),
    )(idxs.reshape(_T, 1), sr.reshape(_T, 1), sf.reshape(_T, 1),
      x_sorted, rg, ru, fg, fu, rdv, fdv)


def kernel(x, token_lora_indices, base_gate_w, base_up_w, base_down_w,
           retain_gate_stacked, retain_up_stacked, retain_down_stacked,
           forget_gate_stacked, forget_up_stacked, forget_down_stacked,
           scales):
    idx0 = jnp.maximum(token_lora_indices, 0)
    inv, pos, idxs, sr, sf, x_sorted = _sc_sort(
        idx0, scales.reshape(_E * 2), x)

    base_out = _tc_base(x, base_gate_w, base_up_w, base_down_w)

    rg = retain_gate_stacked.reshape(_E * _NR, _HID)
    ru = retain_up_stacked.reshape(_E * _NR, _HID)
    fg = forget_gate_stacked.reshape(_E * _NF, _HID)
    fu = forget_up_stacked.reshape(_E * _NF, _HID)
    rdv = retain_down_stacked.reshape(_E, _HID, _NR)
    fdv = forget_down_stacked.reshape(_E, _HID, _NF)

    ad_sorted = _tc_adapters(idxs, sr, sf, x_sorted,
                             rg, ru, fg, fu, rdv, fdv)

    return _sc_unsort_add(pos, ad_sorted, base_out)


# speedup vs baseline: 1.1218x; 1.1218x over previous
import jax
import jax.numpy as jnp
from jax import lax
from jax.experimental import pallas as pl
from jax.experimental.pallas import tpu as pltpu
from jax.experimental.pallas import tpu_sc as plsc

_HID = 1024
_INTER = 4096
_E = 64
_NR = 32
_NF = 32
_T = 2048

_CB = 256
_NB_BASE = _INTER // _CB
_NB_AD = (_E * _NR) // _CB
_EPB = _CB // _NR

_TT = 256
_NTT = _T // _TT

_NC = 2
_NS = 16


def _sc_sort_body(idx_hbm, scales_hbm, x_hbm,
                  inv_hbm, pos_hbm, idxs_hbm, sr_hbm, sf_hbm, xs_hbm,
                  idx_v, tok_v, pos_v, post_v, idxs_v, sr_v, sf_v, scales_v,
                  allh_v, cnt_v, off_v, buf_v, buf2_v, myinv_v, rows_v,
                  sh_hist, sh_inv, sh_idxs, sh_sr, sh_sf,
                  sem):
    core = lax.axis_index("c")
    sid = lax.axis_index("s")
    base = sid * 128
    l16 = lax.iota(jnp.int32, 16)

    pltpu.sync_copy(idx_hbm.at[pl.ds(base, 128)], idx_v)
    pltpu.sync_copy(scales_hbm, scales_v)

    def runs(c):
        k16 = idx_v[pl.ds(16 * c, 16)]
        ks, vs = plsc.sort_key_val(k16, l16)
        buf_v[pl.ds(0, 16)] = jnp.full((16,), -1, jnp.int32)
        buf_v[pl.ds(1, 16)] = ks
        prev = buf_v[pl.ds(0, 16)]
        buf2_v[pl.ds(1, 16)] = jnp.full((16,), -2, jnp.int32)
        buf2_v[pl.ds(0, 16)] = ks
        nxt = buf2_v[pl.ds(1, 16)]
        is_new = ks != prev
        last = ks != nxt
        run_start = plsc.cummax(l16, mask=is_new)
        rank = l16 - run_start
        return ks, vs, rank, last

    for k in range(_E // 16):
        cnt_v[pl.ds(16 * k, 16)] = jnp.zeros((16,), jnp.int32)
    for c in range(8):
        ks, vs, rank, last = runs(c)
        cur = plsc.load_gather(cnt_v, [ks])
        plsc.store_scatter(cnt_v, [ks], cur + rank + 1, mask=last)

    pltpu.sync_copy(cnt_v, sh_hist.at[sid])
    plsc.subcore_barrier()
    pltpu.sync_copy(sh_hist, allh_v)

    sid16 = jnp.full((16,), 0, jnp.int32) + sid
    carry = jnp.int32(0)
    for k in range(_E // 16):
        tot16 = jnp.zeros((16,), jnp.int32)
        mine16 = jnp.zeros((16,), jnp.int32)
        for w in range(_NS):
            row = allh_v[w, pl.ds(16 * k, 16)]
            tot16 = tot16 + row
            wlt = jnp.full((16,), w, jnp.int32) < sid16
            mine16 = mine16 + jnp.where(wlt, row, 0)
        gbase16 = carry + plsc.cumsum(tot16) - tot16
        off_v[pl.ds(16 * k, 16)] = gbase16 + mine16
        carry = carry + jnp.sum(tot16)

    for c in range(8):
        ks, vs, rank, last = runs(c)
        offs = plsc.load_gather(off_v, [ks])
        pos16 = offs + rank
        plsc.store_scatter(off_v, [ks], pos16 + 1, mask=last)
        pos_v[pl.ds(16 * c, 16)] = pos16
        tok_v[pl.ds(16 * c, 16)] = base + 16 * c + vs
        idxs_v[pl.ds(16 * c, 16)] = ks
        sr_v[pl.ds(16 * c, 16)] = plsc.load_gather(scales_v, [2 * ks])
        sf_v[pl.ds(16 * c, 16)] = plsc.load_gather(scales_v, [2 * ks + 1])
        plsc.store_scatter(post_v, [16 * c + vs], pos16)

    pltpu.async_copy(tok_v, sh_inv.at[pos_v], sem).wait()
    pltpu.async_copy(idxs_v, sh_idxs.at[pos_v], sem).wait()
    pltpu.async_copy(sr_v, sh_sr.at[pos_v], sem).wait()
    pltpu.async_copy(sf_v, sh_sf.at[pos_v], sem).wait()
    plsc.subcore_barrier()

    @pl.when(core == 0)
    def _():
        pltpu.sync_copy(sh_inv.at[pl.ds(base, 128)],
                        inv_hbm.at[pl.ds(base, 128)])
        pltpu.sync_copy(sh_idxs.at[pl.ds(base, 128)],
                        idxs_hbm.at[pl.ds(base, 128)])
        pltpu.sync_copy(sh_sr.at[pl.ds(base, 128)],
                        sr_hbm.at[pl.ds(base, 128)])
        pltpu.sync_copy(sh_sf.at[pl.ds(base, 128)],
                        sf_hbm.at[pl.ds(base, 128)])
        pltpu.sync_copy(post_v, pos_hbm.at[pl.ds(base, 128)])

    row0 = core * 1024 + sid * 64
    pltpu.sync_copy(sh_inv.at[pl.ds(row0, 64)], myinv_v)
    for j in range(4):
        pltpu.async_copy(x_hbm.at[myinv_v.at[pl.ds(16 * j, 16)]],
                         rows_v, sem).wait()
        pltpu.sync_copy(rows_v, xs_hbm.at[pl.ds(row0 + 16 * j, 16)])


def _sc_sort(idx, scales_flat, x):
    mesh = plsc.VectorSubcoreMesh(core_axis_name="c", subcore_axis_name="s")
    f = pl.kernel(
        _sc_sort_body,
        mesh=mesh,
        out_type=(
            jax.ShapeDtypeStruct((_T,), jnp.int32),
            jax.ShapeDtypeStruct((_T,), jnp.int32),
            jax.ShapeDtypeStruct((_T,), jnp.int32),
            jax.ShapeDtypeStruct((_T,), jnp.float32),
            jax.ShapeDtypeStruct((_T,), jnp.float32),
            jax.ShapeDtypeStruct((_T, _HID), jnp.float32),
        ),
        scratch_types=[
            pltpu.VMEM((128,), jnp.int32),
            pltpu.VMEM((128,), jnp.int32),
            pltpu.VMEM((128,), jnp.int32),
            pltpu.VMEM((128,), jnp.int32),
            pltpu.VMEM((128,), jnp.int32),
            pltpu.VMEM((128,), jnp.float32),
            pltpu.VMEM((128,), jnp.float32),
            pltpu.VMEM((128,), jnp.float32),
            pltpu.VMEM((_NS, _E), jnp.int32),
            pltpu.VMEM((_E,), jnp.int32),
            pltpu.VMEM((_E,), jnp.int32),
            pltpu.VMEM((32,), jnp.int32),
            pltpu.VMEM((32,), jnp.int32),
            pltpu.VMEM((64,), jnp.int32),
            pltpu.VMEM((16, _HID), jnp.float32),
            pltpu.VMEM_SHARED((_NS, _E), jnp.int32),
            pltpu.VMEM_SHARED((_T,), jnp.int32),
            pltpu.VMEM_SHARED((_T,), jnp.int32),
            pltpu.VMEM_SHARED((_T,), jnp.float32),
            pltpu.VMEM_SHARED((_T,), jnp.float32),
            pltpu.SemaphoreType.DMA,
        ],
        compiler_params=pltpu.CompilerParams(needs_layout_passes=False),
    )
    return f(idx, scales_flat, x)


def _sc_unsort_add_body(pos_hbm, ads_hbm, base_hbm, out_hbm,
                        pos_v, rows_v, base_v, sem):
    core = lax.axis_index("c")
    sid = lax.axis_index("s")
    row0 = core * 1024 + sid * 64
    pltpu.sync_copy(pos_hbm.at[pl.ds(row0, 64)], pos_v)
    for j in range(4):
        cp = pltpu.async_copy(
            ads_hbm.at[pos_v.at[pl.ds(16 * j, 16)]], rows_v, sem)
        pltpu.sync_copy(base_hbm.at[pl.ds(row0 + 16 * j, 16)], base_v)
        cp.wait()

        def addk(k, carry):
            s = pl.ds(k * 16, 16)
            for r in range(16):
                rows_v[r, s] = rows_v[r, s] + base_v[r, s]
            return carry

        lax.fori_loop(0, _HID // 16, addk, jnp.int32(0))
        pltpu.sync_copy(rows_v, out_hbm.at[pl.ds(row0 + 16 * j, 16)])


def _sc_unsort_add(pos, ad_sorted, base_out):
    mesh = plsc.VectorSubcoreMesh(core_axis_name="c", subcore_axis_name="s")
    f = pl.kernel(
        _sc_unsort_add_body,
        mesh=mesh,
        out_type=jax.ShapeDtypeStruct((_T, _HID), jnp.float32),
        scratch_types=[
            pltpu.VMEM((64,), jnp.int32),
            pltpu.VMEM((16, _HID), jnp.float32),
            pltpu.VMEM((16, _HID), jnp.float32),
            pltpu.SemaphoreType.DMA,
        ],
    )
    return f(pos, ad_sorted, base_out)


def _base_body(x_ref, bg_ref, bu_ref, bd_ref, out_ref, wg_s, wu_s, bd_s):
    c = pl.program_id(0)
    t = pl.program_id(1)

    @pl.when(t == 0)
    def _():
        wg_s[...] = bg_ref[...].astype(jnp.bfloat16)
        wu_s[...] = bu_ref[...].astype(jnp.bfloat16)
        bd_s[...] = bd_ref[...].astype(jnp.bfloat16)

    rows = pl.ds(t * _TT, _TT)
    x = x_ref[rows, :]
    dn = (((1,), (1,)), ((), ()))
    g = lax.dot_general(x, wg_s[...], dn, preferred_element_type=jnp.float32)
    u = lax.dot_general(x, wu_s[...], dn, preferred_element_type=jnp.float32)
    sig = 1.0 / (1.0 + jnp.exp(-g))
    h = (g * sig) * u
    contrib = lax.dot_general(h.astype(jnp.bfloat16), bd_s[...], dn,
                              preferred_element_type=jnp.float32)

    @pl.when(c == 0)
    def _():
        out_ref[rows, :] = contrib

    @pl.when(c > 0)
    def _():
        out_ref[rows, :] += contrib


def _tc_base(xb, base_gate_w, base_up_w, base_down_w):
    return pl.pallas_call(
        _base_body,
        grid=(_NB_BASE, _NTT),
        in_specs=[
            pl.BlockSpec((_T, _HID), lambda c, t: (0, 0)),
            pl.BlockSpec((_CB, _HID), lambda c, t: (c, 0)),
            pl.BlockSpec((_CB, _HID), lambda c, t: (c, 0)),
            pl.BlockSpec((_HID, _CB), lambda c, t: (0, c)),
        ],
        out_specs=pl.BlockSpec((_T, _HID), lambda c, t: (0, 0)),
        out_shape=jax.ShapeDtypeStruct((_T, _HID), jnp.float32),
        scratch_shapes=[
            pltpu.VMEM((_CB, _HID), jnp.bfloat16),
            pltpu.VMEM((_CB, _HID), jnp.bfloat16),
            pltpu.VMEM((_HID, _CB), jnp.bfloat16),
        ],
        compiler_params=pltpu.CompilerParams(
            dimension_semantics=("arbitrary", "arbitrary")),
    )(xb, base_gate_w, base_up_w, base_down_w)


def _ad_body(idx_ref, sr_ref, sf_ref, xs_ref, rg_ref, ru_ref, fg_ref, fu_ref,
             rd_ref, fd_ref, out_ref, wg_s, wu_s, wd_s):
    c = pl.program_id(0)
    t = pl.program_id(1)
    rows = pl.ds(t * _TT, _TT)

    @pl.when(t == 0)
    def _():
        @pl.when(c < _NB_AD)
        def _():
            wg_s[...] = rg_ref[...].astype(jnp.bfloat16)
            wu_s[...] = ru_ref[...].astype(jnp.bfloat16)
            wd_s[...] = rd_ref[...]

        @pl.when(c >= _NB_AD)
        def _():
            wg_s[...] = fg_ref[...].astype(jnp.bfloat16)
            wu_s[...] = fu_ref[...].astype(jnp.bfloat16)
            wd_s[...] = fd_ref[...]

    @pl.when(c == 0)
    def _():
        out_ref[rows, :] = jnp.zeros((_TT, _HID), jnp.float32)

    blk = jnp.where(c < _NB_AD, c, c - _NB_AD)
    e0 = blk * _EPB
    idxv = idx_ref[rows, :]
    emin = jnp.min(idxv)
    emax = jnp.max(idxv)

    def run(s_ref):
        x = xs_ref[rows, :]
        dn = (((1,), (1,)), ((), ()))
        g = lax.dot_general(x, wg_s[...], dn,
                            preferred_element_type=jnp.float32)
        u = lax.dot_general(x, wu_s[...], dn,
                            preferred_element_type=jnp.float32)
        sig = 1.0 / (1.0 + jnp.exp(-g))
        h = (g * sig) * u
        ecol = e0 + lax.broadcasted_iota(jnp.int32, (_TT, _CB), 1) // _NR
        h = jnp.where(ecol == idxv, h * s_ref[rows, :], 0.0)
        contrib = jnp.dot(h.astype(jnp.bfloat16), wd_s[...],
                          preferred_element_type=jnp.float32)
        out_ref[rows, :] += contrib

    @pl.when((emax >= e0) & (emin < e0 + _EPB) & (c < _NB_AD))
    def _():
        run(sr_ref)

    @pl.when((emax >= e0) & (emin < e0 + _EPB) & (c >= _NB_AD))
    def _():
        run(sf_ref)


def _tc_adapters(idxs, sr, sf, xs, rg, ru, fg, fu, rd, fd):
    nb = _NB_AD

    def rblk(c, t):
        return (jnp.clip(c, 0, nb - 1), 0)

    def fblk(c, t):
        return (jnp.clip(c - nb, 0, nb - 1), 0)

    return pl.pallas_call(
        _ad_body,
        grid=(2 * nb, _NTT),
        in_specs=[
            pl.BlockSpec((_T, 1), lambda c, t: (0, 0)),
            pl.BlockSpec((_T, 1), lambda c, t: (0, 0)),
            pl.BlockSpec((_T, 1), lambda c, t: (0, 0)),
            pl.BlockSpec((_T, _HID), lambda c, t: (0, 0)),
            pl.BlockSpec((_CB, _HID), rblk),
            pl.BlockSpec((_CB, _HID), rblk),
            pl.BlockSpec((_CB, _HID), fblk),
            pl.BlockSpec((_CB, _HID), fblk),
            pl.BlockSpec((_CB, _HID), rblk),
            pl.BlockSpec((_CB, _HID), fblk),
        ],
        out_specs=pl.BlockSpec((_T, _HID), lambda c, t: (0, 0)),
        out_shape=jax.ShapeDtypeStruct((_T, _HID), jnp.float32),
        scratch_shapes=[
            pltpu.VMEM((_CB, _HID), jnp.bfloat16),
            pltpu.VMEM((_CB, _HID), jnp.bfloat16),
            pltpu.VMEM((_CB, _HID), jnp.bfloat16),
        ],
        compiler_params=pltpu.CompilerParams(
            dimension_semantics=("arbitrary", "arbitrary")),
    )(idxs.reshape(_T, 1), sr.reshape(_T, 1), sf.reshape(_T, 1),
      xs, rg, ru, fg, fu, rd, fd)


def kernel(x, token_lora_indices, base_gate_w, base_up_w, base_down_w,
           retain_gate_stacked, retain_up_stacked, retain_down_stacked,
           forget_gate_stacked, forget_up_stacked, forget_down_stacked,
           scales):
    idx0 = jnp.maximum(token_lora_indices, 0)
    inv, pos, idxs, sr, sf, x_sorted = _sc_sort(
        idx0, scales.reshape(_E * 2), x)

    base_out = _tc_base(x.astype(jnp.bfloat16),
                        base_gate_w, base_up_w, base_down_w)

    rg = retain_gate_stacked.reshape(_E * _NR, _HID)
    ru = retain_up_stacked.reshape(_E * _NR, _HID)
    fg = forget_gate_stacked.reshape(_E * _NF, _HID)
    fu = forget_up_stacked.reshape(_E * _NF, _HID)
    rd = retain_down_stacked[:, 0].transpose(0, 2, 1).reshape(
        _E * _NR, _HID).astype(jnp.bfloat16)
    fd = forget_down_stacked[:, 0].transpose(0, 2, 1).reshape(
        _E * _NF, _HID).astype(jnp.bfloat16)

    ad_sorted = _tc_adapters(idxs, sr, sf, x_sorted.astype(jnp.bfloat16),
                             rg, ru, fg, fu, rd, fd)

    return _sc_unsort_add(pos, ad_sorted, base_out)


# speedup vs baseline: 1.4025x; 1.2502x over previous
import jax
import jax.numpy as jnp
from jax import lax
from jax.experimental import pallas as pl
from jax.experimental.pallas import tpu as pltpu
from jax.experimental.pallas import tpu_sc as plsc

_HID = 1024
_INTER = 4096
_E = 64
_NR = 32
_NF = 32
_T = 2048

_CB = 512
_NB_BASE = _INTER // _CB
_NB_AD = (_E * _NR) // _CB
_EPB = _CB // _NR

_TT = 256
_NTT = _T // _TT

_NC = 2
_NS = 16


def _sc_sort_body(idx_hbm, scales_hbm, x_hbm,
                  inv_hbm, pos_hbm, idxs_hbm, sr_hbm, sf_hbm, xs_hbm,
                  idx_v, tok_v, pos_v, post_v, idxs_v, sr_v, sf_v, scales_v,
                  allh_v, cnt_v, off_v, buf_v, buf2_v, myinv_v, rows_v,
                  sh_hist, sh_inv, sh_idxs, sh_sr, sh_sf,
                  sem):
    core = lax.axis_index("c")
    sid = lax.axis_index("s")
    base = sid * 128
    l16 = lax.iota(jnp.int32, 16)

    pltpu.sync_copy(idx_hbm.at[pl.ds(base, 128)], idx_v)
    pltpu.sync_copy(scales_hbm, scales_v)

    def runs(c):
        k16 = idx_v[pl.ds(16 * c, 16)]
        ks, vs = plsc.sort_key_val(k16, l16)
        buf_v[pl.ds(0, 16)] = jnp.full((16,), -1, jnp.int32)
        buf_v[pl.ds(1, 16)] = ks
        prev = buf_v[pl.ds(0, 16)]
        buf2_v[pl.ds(1, 16)] = jnp.full((16,), -2, jnp.int32)
        buf2_v[pl.ds(0, 16)] = ks
        nxt = buf2_v[pl.ds(1, 16)]
        is_new = ks != prev
        last = ks != nxt
        run_start = plsc.cummax(l16, mask=is_new)
        rank = l16 - run_start
        return ks, vs, rank, last

    for k in range(_E // 16):
        cnt_v[pl.ds(16 * k, 16)] = jnp.zeros((16,), jnp.int32)
    for c in range(8):
        ks, vs, rank, last = runs(c)
        cur = plsc.load_gather(cnt_v, [ks])
        plsc.store_scatter(cnt_v, [ks], cur + rank + 1, mask=last)

    pltpu.sync_copy(cnt_v, sh_hist.at[sid])
    plsc.subcore_barrier()
    pltpu.sync_copy(sh_hist, allh_v)

    sid16 = jnp.full((16,), 0, jnp.int32) + sid
    carry = jnp.int32(0)
    for k in range(_E // 16):
        tot16 = jnp.zeros((16,), jnp.int32)
        mine16 = jnp.zeros((16,), jnp.int32)
        for w in range(_NS):
            row = allh_v[w, pl.ds(16 * k, 16)]
            tot16 = tot16 + row
            wlt = jnp.full((16,), w, jnp.int32) < sid16
            mine16 = mine16 + jnp.where(wlt, row, 0)
        gbase16 = carry + plsc.cumsum(tot16) - tot16
        off_v[pl.ds(16 * k, 16)] = gbase16 + mine16
        carry = carry + jnp.sum(tot16)

    for c in range(8):
        ks, vs, rank, last = runs(c)
        offs = plsc.load_gather(off_v, [ks])
        pos16 = offs + rank
        plsc.store_scatter(off_v, [ks], pos16 + 1, mask=last)
        pos_v[pl.ds(16 * c, 16)] = pos16
        tok_v[pl.ds(16 * c, 16)] = base + 16 * c + vs
        idxs_v[pl.ds(16 * c, 16)] = ks
        sr_v[pl.ds(16 * c, 16)] = plsc.load_gather(scales_v, [2 * ks])
        sf_v[pl.ds(16 * c, 16)] = plsc.load_gather(scales_v, [2 * ks + 1])
        plsc.store_scatter(post_v, [16 * c + vs], pos16)

    pltpu.async_copy(tok_v, sh_inv.at[pos_v], sem).wait()
    pltpu.async_copy(idxs_v, sh_idxs.at[pos_v], sem).wait()
    pltpu.async_copy(sr_v, sh_sr.at[pos_v], sem).wait()
    pltpu.async_copy(sf_v, sh_sf.at[pos_v], sem).wait()
    plsc.subcore_barrier()

    @pl.when(core == 0)
    def _():
        pltpu.sync_copy(sh_inv.at[pl.ds(base, 128)],
                        inv_hbm.at[pl.ds(base, 128)])
        pltpu.sync_copy(sh_idxs.at[pl.ds(base, 128)],
                        idxs_hbm.at[pl.ds(base, 128)])
        pltpu.sync_copy(sh_sr.at[pl.ds(base, 128)],
                        sr_hbm.at[pl.ds(base, 128)])
        pltpu.sync_copy(sh_sf.at[pl.ds(base, 128)],
                        sf_hbm.at[pl.ds(base, 128)])
        pltpu.sync_copy(post_v, pos_hbm.at[pl.ds(base, 128)])

    row0 = core * 1024 + sid * 64
    pltpu.sync_copy(sh_inv.at[pl.ds(row0, 64)], myinv_v)
    for j in range(4):
        pltpu.async_copy(x_hbm.at[myinv_v.at[pl.ds(16 * j, 16)]],
                         rows_v, sem).wait()
        pltpu.sync_copy(rows_v, xs_hbm.at[pl.ds(row0 + 16 * j, 16)])


def _sc_sort(idx, scales_flat, x):
    mesh = plsc.VectorSubcoreMesh(core_axis_name="c", subcore_axis_name="s")
    f = pl.kernel(
        _sc_sort_body,
        mesh=mesh,
        out_type=(
            jax.ShapeDtypeStruct((_T,), jnp.int32),
            jax.ShapeDtypeStruct((_T,), jnp.int32),
            jax.ShapeDtypeStruct((_T,), jnp.int32),
            jax.ShapeDtypeStruct((_T,), jnp.float32),
            jax.ShapeDtypeStruct((_T,), jnp.float32),
            jax.ShapeDtypeStruct((_T, _HID), jnp.float32),
        ),
        scratch_types=[
            pltpu.VMEM((128,), jnp.int32),
            pltpu.VMEM((128,), jnp.int32),
            pltpu.VMEM((128,), jnp.int32),
            pltpu.VMEM((128,), jnp.int32),
            pltpu.VMEM((128,), jnp.int32),
            pltpu.VMEM((128,), jnp.float32),
            pltpu.VMEM((128,), jnp.float32),
            pltpu.VMEM((128,), jnp.float32),
            pltpu.VMEM((_NS, _E), jnp.int32),
            pltpu.VMEM((_E,), jnp.int32),
            pltpu.VMEM((_E,), jnp.int32),
            pltpu.VMEM((32,), jnp.int32),
            pltpu.VMEM((32,), jnp.int32),
            pltpu.VMEM((64,), jnp.int32),
            pltpu.VMEM((16, _HID), jnp.float32),
            pltpu.VMEM_SHARED((_NS, _E), jnp.int32),
            pltpu.VMEM_SHARED((_T,), jnp.int32),
            pltpu.VMEM_SHARED((_T,), jnp.int32),
            pltpu.VMEM_SHARED((_T,), jnp.float32),
            pltpu.VMEM_SHARED((_T,), jnp.float32),
            pltpu.SemaphoreType.DMA,
        ],
        compiler_params=pltpu.CompilerParams(needs_layout_passes=False),
    )
    return f(idx, scales_flat, x)


def _sc_unsort_add_body(pos_hbm, ads_hbm, base_hbm, out_hbm,
                        pos_v, rows_v, base_v, sem):
    core = lax.axis_index("c")
    sid = lax.axis_index("s")
    row0 = core * 1024 + sid * 64
    pltpu.sync_copy(pos_hbm.at[pl.ds(row0, 64)], pos_v)
    for j in range(4):
        cp = pltpu.async_copy(
            ads_hbm.at[pos_v.at[pl.ds(16 * j, 16)]], rows_v, sem)
        pltpu.sync_copy(base_hbm.at[pl.ds(row0 + 16 * j, 16)], base_v)
        cp.wait()

        def addk(k, carry):
            s = pl.ds(k * 16, 16)
            for r in range(16):
                rows_v[r, s] = rows_v[r, s] + base_v[r, s]
            return carry

        lax.fori_loop(0, _HID // 16, addk, jnp.int32(0))
        pltpu.sync_copy(rows_v, out_hbm.at[pl.ds(row0 + 16 * j, 16)])


def _sc_unsort_add(pos, ad_sorted, base_out):
    mesh = plsc.VectorSubcoreMesh(core_axis_name="c", subcore_axis_name="s")
    f = pl.kernel(
        _sc_unsort_add_body,
        mesh=mesh,
        out_type=jax.ShapeDtypeStruct((_T, _HID), jnp.float32),
        scratch_types=[
            pltpu.VMEM((64,), jnp.int32),
            pltpu.VMEM((16, _HID), jnp.float32),
            pltpu.VMEM((16, _HID), jnp.float32),
            pltpu.SemaphoreType.DMA,
        ],
    )
    return f(pos, ad_sorted, base_out)


def _base_body(x_ref, bg_ref, bu_ref, bd_ref, out_ref, wg_s, wu_s, bd_s):
    c = pl.program_id(0)
    t = pl.program_id(1)

    @pl.when(t == 0)
    def _():
        wg_s[...] = bg_ref[...].astype(jnp.bfloat16)
        wu_s[...] = bu_ref[...].astype(jnp.bfloat16)
        bd_s[...] = bd_ref[...].astype(jnp.bfloat16)

    rows = pl.ds(t * _TT, _TT)
    x = x_ref[rows, :]
    dn = (((1,), (1,)), ((), ()))
    g = lax.dot_general(x, wg_s[...], dn, preferred_element_type=jnp.float32)
    u = lax.dot_general(x, wu_s[...], dn, preferred_element_type=jnp.float32)
    sig = 1.0 / (1.0 + jnp.exp(-g))
    h = (g * sig) * u
    contrib = lax.dot_general(h.astype(jnp.bfloat16), bd_s[...], dn,
                              preferred_element_type=jnp.float32)

    @pl.when(c == 0)
    def _():
        out_ref[rows, :] = contrib

    @pl.when(c > 0)
    def _():
        out_ref[rows, :] += contrib


def _tc_base(xb, base_gate_w, base_up_w, base_down_w):
    return pl.pallas_call(
        _base_body,
        grid=(_NB_BASE, _NTT),
        in_specs=[
            pl.BlockSpec((_T, _HID), lambda c, t: (0, 0)),
            pl.BlockSpec((_CB, _HID), lambda c, t: (c, 0)),
            pl.BlockSpec((_CB, _HID), lambda c, t: (c, 0)),
            pl.BlockSpec((_HID, _CB), lambda c, t: (0, c)),
        ],
        out_specs=pl.BlockSpec((_T, _HID), lambda c, t: (0, 0)),
        out_shape=jax.ShapeDtypeStruct((_T, _HID), jnp.float32),
        scratch_shapes=[
            pltpu.VMEM((_CB, _HID), jnp.bfloat16),
            pltpu.VMEM((_CB, _HID), jnp.bfloat16),
            pltpu.VMEM((_HID, _CB), jnp.bfloat16),
        ],
        compiler_params=pltpu.CompilerParams(
            dimension_semantics=("arbitrary", "arbitrary")),
    )(xb, base_gate_w, base_up_w, base_down_w)


def _ad_body(idx_ref, sr_ref, sf_ref, xs_ref, rg_ref, ru_ref, fg_ref, fu_ref,
             rd_ref, fd_ref, out_ref, wg_s, wu_s, wd_s):
    c = pl.program_id(0)
    t = pl.program_id(1)
    rows = pl.ds(t * _TT, _TT)

    @pl.when(t == 0)
    def _():
        @pl.when(c < _NB_AD)
        def _():
            wg_s[...] = rg_ref[...].astype(jnp.bfloat16)
            wu_s[...] = ru_ref[...].astype(jnp.bfloat16)
            wd_s[...] = rd_ref[...]

        @pl.when(c >= _NB_AD)
        def _():
            wg_s[...] = fg_ref[...].astype(jnp.bfloat16)
            wu_s[...] = fu_ref[...].astype(jnp.bfloat16)
            wd_s[...] = fd_ref[...]

    @pl.when(c == 0)
    def _():
        out_ref[rows, :] = jnp.zeros((_TT, _HID), jnp.float32)

    blk = jnp.where(c < _NB_AD, c, c - _NB_AD)
    e0 = blk * _EPB
    idxv = idx_ref[rows, :]
    emin = jnp.min(idxv)
    emax = jnp.max(idxv)

    def run(s_ref):
        x = xs_ref[rows, :]
        dn = (((1,), (1,)), ((), ()))
        g = lax.dot_general(x, wg_s[...], dn,
                            preferred_element_type=jnp.float32)
        u = lax.dot_general(x, wu_s[...], dn,
                            preferred_element_type=jnp.float32)
        sig = 1.0 / (1.0 + jnp.exp(-g))
        h = (g * sig) * u
        ecol = e0 + lax.broadcasted_iota(jnp.int32, (_TT, _CB), 1) // _NR
        h = jnp.where(ecol == idxv, h * s_ref[rows, :], 0.0)
        contrib = jnp.dot(h.astype(jnp.bfloat16), wd_s[...],
                          preferred_element_type=jnp.float32)
        out_ref[rows, :] += contrib

    @pl.when((emax >= e0) & (emin < e0 + _EPB) & (c < _NB_AD))
    def _():
        run(sr_ref)

    @pl.when((emax >= e0) & (emin < e0 + _EPB) & (c >= _NB_AD))
    def _():
        run(sf_ref)


def _tc_adapters(idxs, sr, sf, xs, rg, ru, fg, fu, rd, fd):
    nb = _NB_AD

    def rblk(c, t):
        return (jnp.clip(c, 0, nb - 1), 0)

    def fblk(c, t):
        return (jnp.clip(c - nb, 0, nb - 1), 0)

    return pl.pallas_call(
        _ad_body,
        grid=(2 * nb, _NTT),
        in_specs=[
            pl.BlockSpec((_T, 1), lambda c, t: (0, 0)),
            pl.BlockSpec((_T, 1), lambda c, t: (0, 0)),
            pl.BlockSpec((_T, 1), lambda c, t: (0, 0)),
            pl.BlockSpec((_T, _HID), lambda c, t: (0, 0)),
            pl.BlockSpec((_CB, _HID), rblk),
            pl.BlockSpec((_CB, _HID), rblk),
            pl.BlockSpec((_CB, _HID), fblk),
            pl.BlockSpec((_CB, _HID), fblk),
            pl.BlockSpec((_CB, _HID), rblk),
            pl.BlockSpec((_CB, _HID), fblk),
        ],
        out_specs=pl.BlockSpec((_T, _HID), lambda c, t: (0, 0)),
        out_shape=jax.ShapeDtypeStruct((_T, _HID), jnp.float32),
        scratch_shapes=[
            pltpu.VMEM((_CB, _HID), jnp.bfloat16),
            pltpu.VMEM((_CB, _HID), jnp.bfloat16),
            pltpu.VMEM((_CB, _HID), jnp.bfloat16),
        ],
        compiler_params=pltpu.CompilerParams(
            dimension_semantics=("arbitrary", "arbitrary")),
    )(idxs.reshape(_T, 1), sr.reshape(_T, 1), sf.reshape(_T, 1),
      xs, rg, ru, fg, fu, rd, fd)


def kernel(x, token_lora_indices, base_gate_w, base_up_w, base_down_w,
           retain_gate_stacked, retain_up_stacked, retain_down_stacked,
           forget_gate_stacked, forget_up_stacked, forget_down_stacked,
           scales):
    idx0 = jnp.maximum(token_lora_indices, 0)
    inv, pos, idxs, sr, sf, x_sorted = _sc_sort(
        idx0, scales.reshape(_E * 2), x)

    base_out = _tc_base(x.astype(jnp.bfloat16),
                        base_gate_w, base_up_w, base_down_w)

    rg = retain_gate_stacked.reshape(_E * _NR, _HID)
    ru = retain_up_stacked.reshape(_E * _NR, _HID)
    fg = forget_gate_stacked.reshape(_E * _NF, _HID)
    fu = forget_up_stacked.reshape(_E * _NF, _HID)
    rd = retain_down_stacked[:, 0].transpose(0, 2, 1).reshape(
        _E * _NR, _HID).astype(jnp.bfloat16)
    fd = forget_down_stacked[:, 0].transpose(0, 2, 1).reshape(
        _E * _NF, _HID).astype(jnp.bfloat16)

    ad_sorted = _tc_adapters(idxs, sr, sf, x_sorted.astype(jnp.bfloat16),
                             rg, ru, fg, fu, rd, fd)

    return _sc_unsort_add(pos, ad_sorted, base_out)


# speedup vs baseline: 1.5449x; 1.1015x over previous
import jax
import jax.numpy as jnp
from jax import lax
from jax.experimental import pallas as pl
from jax.experimental.pallas import tpu as pltpu
from jax.experimental.pallas import tpu_sc as plsc

_HID = 1024
_INTER = 4096
_E = 64
_NR = 32
_NF = 32
_T = 2048

_CB = 512
_CBB = 1024
_NB_BASE = _INTER // _CBB
_NB_AD = (_E * _NR) // _CB
_EPB = _CB // _NR

_TT = 256
_NTT = _T // _TT

_NC = 2
_NS = 16


def _sc_sort_body(idx_hbm, scales_hbm, x_hbm,
                  inv_hbm, pos_hbm, idxs_hbm, sr_hbm, sf_hbm, xs_hbm,
                  idx_v, tok_v, pos_v, post_v, idxs_v, sr_v, sf_v, scales_v,
                  allh_v, cnt_v, off_v, buf_v, buf2_v, myinv_v, rows_v,
                  sh_hist, sh_inv, sh_idxs, sh_sr, sh_sf,
                  sem):
    core = lax.axis_index("c")
    sid = lax.axis_index("s")
    base = sid * 128
    l16 = lax.iota(jnp.int32, 16)

    pltpu.sync_copy(idx_hbm.at[pl.ds(base, 128)], idx_v)
    pltpu.sync_copy(scales_hbm, scales_v)

    def runs(c):
        k16 = idx_v[pl.ds(16 * c, 16)]
        ks, vs = plsc.sort_key_val(k16, l16)
        buf_v[pl.ds(0, 16)] = jnp.full((16,), -1, jnp.int32)
        buf_v[pl.ds(1, 16)] = ks
        prev = buf_v[pl.ds(0, 16)]
        buf2_v[pl.ds(1, 16)] = jnp.full((16,), -2, jnp.int32)
        buf2_v[pl.ds(0, 16)] = ks
        nxt = buf2_v[pl.ds(1, 16)]
        is_new = ks != prev
        last = ks != nxt
        run_start = plsc.cummax(l16, mask=is_new)
        rank = l16 - run_start
        return ks, vs, rank, last

    for k in range(_E // 16):
        cnt_v[pl.ds(16 * k, 16)] = jnp.zeros((16,), jnp.int32)
    for c in range(8):
        ks, vs, rank, last = runs(c)
        cur = plsc.load_gather(cnt_v, [ks])
        plsc.store_scatter(cnt_v, [ks], cur + rank + 1, mask=last)

    pltpu.sync_copy(cnt_v, sh_hist.at[sid])
    plsc.subcore_barrier()
    pltpu.sync_copy(sh_hist, allh_v)

    sid16 = jnp.full((16,), 0, jnp.int32) + sid
    carry = jnp.int32(0)
    for k in range(_E // 16):
        tot16 = jnp.zeros((16,), jnp.int32)
        mine16 = jnp.zeros((16,), jnp.int32)
        for w in range(_NS):
            row = allh_v[w, pl.ds(16 * k, 16)]
            tot16 = tot16 + row
            wlt = jnp.full((16,), w, jnp.int32) < sid16
            mine16 = mine16 + jnp.where(wlt, row, 0)
        gbase16 = carry + plsc.cumsum(tot16) - tot16
        off_v[pl.ds(16 * k, 16)] = gbase16 + mine16
        carry = carry + jnp.sum(tot16)

    for c in range(8):
        ks, vs, rank, last = runs(c)
        offs = plsc.load_gather(off_v, [ks])
        pos16 = offs + rank
        plsc.store_scatter(off_v, [ks], pos16 + 1, mask=last)
        pos_v[pl.ds(16 * c, 16)] = pos16
        tok_v[pl.ds(16 * c, 16)] = base + 16 * c + vs
        idxs_v[pl.ds(16 * c, 16)] = ks
        sr_v[pl.ds(16 * c, 16)] = plsc.load_gather(scales_v, [2 * ks])
        sf_v[pl.ds(16 * c, 16)] = plsc.load_gather(scales_v, [2 * ks + 1])
        plsc.store_scatter(post_v, [16 * c + vs], pos16)

    pltpu.async_copy(tok_v, sh_inv.at[pos_v], sem).wait()
    pltpu.async_copy(idxs_v, sh_idxs.at[pos_v], sem).wait()
    pltpu.async_copy(sr_v, sh_sr.at[pos_v], sem).wait()
    pltpu.async_copy(sf_v, sh_sf.at[pos_v], sem).wait()
    plsc.subcore_barrier()

    @pl.when(core == 0)
    def _():
        pltpu.sync_copy(sh_inv.at[pl.ds(base, 128)],
                        inv_hbm.at[pl.ds(base, 128)])
        pltpu.sync_copy(sh_idxs.at[pl.ds(base, 128)],
                        idxs_hbm.at[pl.ds(base, 128)])
        pltpu.sync_copy(sh_sr.at[pl.ds(base, 128)],
                        sr_hbm.at[pl.ds(base, 128)])
        pltpu.sync_copy(sh_sf.at[pl.ds(base, 128)],
                        sf_hbm.at[pl.ds(base, 128)])
        pltpu.sync_copy(post_v, pos_hbm.at[pl.ds(base, 128)])

    row0 = core * 1024 + sid * 64
    pltpu.sync_copy(sh_inv.at[pl.ds(row0, 64)], myinv_v)
    for j in range(4):
        pltpu.async_copy(x_hbm.at[myinv_v.at[pl.ds(16 * j, 16)]],
                         rows_v, sem).wait()
        pltpu.sync_copy(rows_v, xs_hbm.at[pl.ds(row0 + 16 * j, 16)])


def _sc_sort(idx, scales_flat, x):
    mesh = plsc.VectorSubcoreMesh(core_axis_name="c", subcore_axis_name="s")
    f = pl.kernel(
        _sc_sort_body,
        mesh=mesh,
        out_type=(
            jax.ShapeDtypeStruct((_T,), jnp.int32),
            jax.ShapeDtypeStruct((_T,), jnp.int32),
            jax.ShapeDtypeStruct((_T,), jnp.int32),
            jax.ShapeDtypeStruct((_T,), jnp.float32),
            jax.ShapeDtypeStruct((_T,), jnp.float32),
            jax.ShapeDtypeStruct((_T, _HID), jnp.float32),
        ),
        scratch_types=[
            pltpu.VMEM((128,), jnp.int32),
            pltpu.VMEM((128,), jnp.int32),
            pltpu.VMEM((128,), jnp.int32),
            pltpu.VMEM((128,), jnp.int32),
            pltpu.VMEM((128,), jnp.int32),
            pltpu.VMEM((128,), jnp.float32),
            pltpu.VMEM((128,), jnp.float32),
            pltpu.VMEM((128,), jnp.float32),
            pltpu.VMEM((_NS, _E), jnp.int32),
            pltpu.VMEM((_E,), jnp.int32),
            pltpu.VMEM((_E,), jnp.int32),
            pltpu.VMEM((32,), jnp.int32),
            pltpu.VMEM((32,), jnp.int32),
            pltpu.VMEM((64,), jnp.int32),
            pltpu.VMEM((16, _HID), jnp.float32),
            pltpu.VMEM_SHARED((_NS, _E), jnp.int32),
            pltpu.VMEM_SHARED((_T,), jnp.int32),
            pltpu.VMEM_SHARED((_T,), jnp.int32),
            pltpu.VMEM_SHARED((_T,), jnp.float32),
            pltpu.VMEM_SHARED((_T,), jnp.float32),
            pltpu.SemaphoreType.DMA,
        ],
        compiler_params=pltpu.CompilerParams(needs_layout_passes=False),
    )
    return f(idx, scales_flat, x)


def _sc_unsort_add_body(pos_hbm, ads_hbm, base_hbm, out_hbm,
                        pos_v, rows_v, base_v, sem):
    core = lax.axis_index("c")
    sid = lax.axis_index("s")
    row0 = core * 1024 + sid * 64
    pltpu.sync_copy(pos_hbm.at[pl.ds(row0, 64)], pos_v)
    for j in range(4):
        cp = pltpu.async_copy(
            ads_hbm.at[pos_v.at[pl.ds(16 * j, 16)]], rows_v, sem)
        pltpu.sync_copy(base_hbm.at[pl.ds(row0 + 16 * j, 16)], base_v)
        cp.wait()

        def addk(k, carry):
            s = pl.ds(k * 16, 16)
            for r in range(16):
                rows_v[r, s] = rows_v[r, s] + base_v[r, s]
            return carry

        lax.fori_loop(0, _HID // 16, addk, jnp.int32(0))
        pltpu.sync_copy(rows_v, out_hbm.at[pl.ds(row0 + 16 * j, 16)])


def _sc_unsort_add(pos, ad_sorted, base_out):
    mesh = plsc.VectorSubcoreMesh(core_axis_name="c", subcore_axis_name="s")
    f = pl.kernel(
        _sc_unsort_add_body,
        mesh=mesh,
        out_type=jax.ShapeDtypeStruct((_T, _HID), jnp.float32),
        scratch_types=[
            pltpu.VMEM((64,), jnp.int32),
            pltpu.VMEM((16, _HID), jnp.float32),
            pltpu.VMEM((16, _HID), jnp.float32),
            pltpu.SemaphoreType.DMA,
        ],
    )
    return f(pos, ad_sorted, base_out)


def _base_body(x_ref, bg_ref, bu_ref, bd_ref, out_ref, wg_s, wu_s, bd_s):
    c = pl.program_id(0)
    t = pl.program_id(1)

    @pl.when(t == 0)
    def _():
        wg_s[...] = bg_ref[...].astype(jnp.bfloat16)
        wu_s[...] = bu_ref[...].astype(jnp.bfloat16)
        bd_s[...] = bd_ref[...].astype(jnp.bfloat16)

    rows = pl.ds(t * _TT, _TT)
    x = x_ref[rows, :]
    dn = (((1,), (1,)), ((), ()))
    g = lax.dot_general(x, wg_s[...], dn, preferred_element_type=jnp.float32)
    u = lax.dot_general(x, wu_s[...], dn, preferred_element_type=jnp.float32)
    sig = 1.0 / (1.0 + jnp.exp(-g))
    h = (g * sig) * u
    contrib = lax.dot_general(h.astype(jnp.bfloat16), bd_s[...], dn,
                              preferred_element_type=jnp.float32)

    @pl.when(c == 0)
    def _():
        out_ref[rows, :] = contrib

    @pl.when(c > 0)
    def _():
        out_ref[rows, :] += contrib


def _tc_base(xb, base_gate_w, base_up_w, base_down_w):
    return pl.pallas_call(
        _base_body,
        grid=(_NB_BASE, _NTT),
        in_specs=[
            pl.BlockSpec((_T, _HID), lambda c, t: (0, 0)),
            pl.BlockSpec((_CBB, _HID), lambda c, t: (c, 0)),
            pl.BlockSpec((_CBB, _HID), lambda c, t: (c, 0)),
            pl.BlockSpec((_HID, _CBB), lambda c, t: (0, c)),
        ],
        out_specs=pl.BlockSpec((_T, _HID), lambda c, t: (0, 0)),
        out_shape=jax.ShapeDtypeStruct((_T, _HID), jnp.float32),
        scratch_shapes=[
            pltpu.VMEM((_CBB, _HID), jnp.bfloat16),
            pltpu.VMEM((_CBB, _HID), jnp.bfloat16),
            pltpu.VMEM((_HID, _CBB), jnp.bfloat16),
        ],
        compiler_params=pltpu.CompilerParams(
            dimension_semantics=("arbitrary", "arbitrary")),
    )(xb, base_gate_w, base_up_w, base_down_w)


def _ad_body(idx_ref, sr_ref, sf_ref, xs_ref, rg_ref, ru_ref, fg_ref, fu_ref,
             rd_ref, fd_ref, out_ref, wg_s, wu_s, wd_s):
    c = pl.program_id(0)
    t = pl.program_id(1)
    rows = pl.ds(t * _TT, _TT)

    @pl.when(t == 0)
    def _():
        @pl.when(c < _NB_AD)
        def _():
            wg_s[...] = rg_ref[...].astype(jnp.bfloat16)
            wu_s[...] = ru_ref[...].astype(jnp.bfloat16)
            wd_s[...] = rd_ref[...]

        @pl.when(c >= _NB_AD)
        def _():
            wg_s[...] = fg_ref[...].astype(jnp.bfloat16)
            wu_s[...] = fu_ref[...].astype(jnp.bfloat16)
            wd_s[...] = fd_ref[...]

    @pl.when(c == 0)
    def _():
        out_ref[rows, :] = jnp.zeros((_TT, _HID), jnp.float32)

    blk = jnp.where(c < _NB_AD, c, c - _NB_AD)
    e0 = blk * _EPB
    idxv = idx_ref[rows, :]
    emin = jnp.min(idxv)
    emax = jnp.max(idxv)

    def run(s_ref):
        x = xs_ref[rows, :]
        dn = (((1,), (1,)), ((), ()))
        g = lax.dot_general(x, wg_s[...], dn,
                            preferred_element_type=jnp.float32)
        u = lax.dot_general(x, wu_s[...], dn,
                            preferred_element_type=jnp.float32)
        sig = 1.0 / (1.0 + jnp.exp(-g))
        h = (g * sig) * u
        ecol = e0 + lax.broadcasted_iota(jnp.int32, (_TT, _CB), 1) // _NR
        h = jnp.where(ecol == idxv, h * s_ref[rows, :], 0.0)
        contrib = jnp.dot(h.astype(jnp.bfloat16), wd_s[...],
                          preferred_element_type=jnp.float32)
        out_ref[rows, :] += contrib

    @pl.when((emax >= e0) & (emin < e0 + _EPB) & (c < _NB_AD))
    def _():
        run(sr_ref)

    @pl.when((emax >= e0) & (emin < e0 + _EPB) & (c >= _NB_AD))
    def _():
        run(sf_ref)


def _tc_adapters(idxs, sr, sf, xs, rg, ru, fg, fu, rd, fd):
    nb = _NB_AD

    def rblk(c, t):
        return (jnp.clip(c, 0, nb - 1), 0)

    def fblk(c, t):
        return (jnp.clip(c - nb, 0, nb - 1), 0)

    return pl.pallas_call(
        _ad_body,
        grid=(2 * nb, _NTT),
        in_specs=[
            pl.BlockSpec((_T, 1), lambda c, t: (0, 0)),
            pl.BlockSpec((_T, 1), lambda c, t: (0, 0)),
            pl.BlockSpec((_T, 1), lambda c, t: (0, 0)),
            pl.BlockSpec((_T, _HID), lambda c, t: (0, 0)),
            pl.BlockSpec((_CB, _HID), rblk),
            pl.BlockSpec((_CB, _HID), rblk),
            pl.BlockSpec((_CB, _HID), fblk),
            pl.BlockSpec((_CB, _HID), fblk),
            pl.BlockSpec((_CB, _HID), rblk),
            pl.BlockSpec((_CB, _HID), fblk),
        ],
        out_specs=pl.BlockSpec((_T, _HID), lambda c, t: (0, 0)),
        out_shape=jax.ShapeDtypeStruct((_T, _HID), jnp.float32),
        scratch_shapes=[
            pltpu.VMEM((_CB, _HID), jnp.bfloat16),
            pltpu.VMEM((_CB, _HID), jnp.bfloat16),
            pltpu.VMEM((_CB, _HID), jnp.bfloat16),
        ],
        compiler_params=pltpu.CompilerParams(
            dimension_semantics=("arbitrary", "arbitrary")),
    )(idxs.reshape(_T, 1), sr.reshape(_T, 1), sf.reshape(_T, 1),
      xs, rg, ru, fg, fu, rd, fd)


def kernel(x, token_lora_indices, base_gate_w, base_up_w, base_down_w,
           retain_gate_stacked, retain_up_stacked, retain_down_stacked,
           forget_gate_stacked, forget_up_stacked, forget_down_stacked,
           scales):
    idx0 = jnp.maximum(token_lora_indices, 0)
    inv, pos, idxs, sr, sf, x_sorted = _sc_sort(
        idx0, scales.reshape(_E * 2), x)

    base_out = _tc_base(x.astype(jnp.bfloat16),
                        base_gate_w, base_up_w, base_down_w)

    rg = retain_gate_stacked.reshape(_E * _NR, _HID)
    ru = retain_up_stacked.reshape(_E * _NR, _HID)
    fg = forget_gate_stacked.reshape(_E * _NF, _HID)
    fu = forget_up_stacked.reshape(_E * _NF, _HID)
    rd = retain_down_stacked[:, 0].transpose(0, 2, 1).reshape(
        _E * _NR, _HID).astype(jnp.bfloat16)
    fd = forget_down_stacked[:, 0].transpose(0, 2, 1).reshape(
        _E * _NF, _HID).astype(jnp.bfloat16)

    ad_sorted = _tc_adapters(idxs, sr, sf, x_sorted.astype(jnp.bfloat16),
                             rg, ru, fg, fu, rd, fd)

    return _sc_unsort_add(pos, ad_sorted, base_out)


# speedup vs baseline: 1.6520x; 1.0693x over previous
import jax
import jax.numpy as jnp
from jax import lax
from jax.experimental import pallas as pl
from jax.experimental.pallas import tpu as pltpu
from jax.experimental.pallas import tpu_sc as plsc

_HID = 1024
_INTER = 4096
_E = 64
_NR = 32
_NF = 32
_T = 2048

_CB = 512
_CBB = 1024
_NB_BASE = _INTER // _CBB
_NB_AD = (_E * _NR) // _CB
_EPB = _CB // _NR

_TT = 512
_NTT = _T // _TT

_NC = 2
_NS = 16


def _sc_sort_body(idx_hbm, scales_hbm, x_hbm,
                  inv_hbm, pos_hbm, idxs_hbm, sr_hbm, sf_hbm, xs_hbm,
                  idx_v, tok_v, pos_v, post_v, idxs_v, sr_v, sf_v, scales_v,
                  allh_v, cnt_v, off_v, buf_v, buf2_v, myinv_v, rows_v,
                  sh_hist, sh_inv, sh_idxs, sh_sr, sh_sf,
                  sem):
    core = lax.axis_index("c")
    sid = lax.axis_index("s")
    base = sid * 128
    l16 = lax.iota(jnp.int32, 16)

    pltpu.sync_copy(idx_hbm.at[pl.ds(base, 128)], idx_v)
    pltpu.sync_copy(scales_hbm, scales_v)

    def runs(c):
        k16 = idx_v[pl.ds(16 * c, 16)]
        ks, vs = plsc.sort_key_val(k16, l16)
        buf_v[pl.ds(0, 16)] = jnp.full((16,), -1, jnp.int32)
        buf_v[pl.ds(1, 16)] = ks
        prev = buf_v[pl.ds(0, 16)]
        buf2_v[pl.ds(1, 16)] = jnp.full((16,), -2, jnp.int32)
        buf2_v[pl.ds(0, 16)] = ks
        nxt = buf2_v[pl.ds(1, 16)]
        is_new = ks != prev
        last = ks != nxt
        run_start = plsc.cummax(l16, mask=is_new)
        rank = l16 - run_start
        return ks, vs, rank, last

    for k in range(_E // 16):
        cnt_v[pl.ds(16 * k, 16)] = jnp.zeros((16,), jnp.int32)
    for c in range(8):
        ks, vs, rank, last = runs(c)
        cur = plsc.load_gather(cnt_v, [ks])
        plsc.store_scatter(cnt_v, [ks], cur + rank + 1, mask=last)

    pltpu.sync_copy(cnt_v, sh_hist.at[sid])
    plsc.subcore_barrier()
    pltpu.sync_copy(sh_hist, allh_v)

    sid16 = jnp.full((16,), 0, jnp.int32) + sid
    carry = jnp.int32(0)
    for k in range(_E // 16):
        tot16 = jnp.zeros((16,), jnp.int32)
        mine16 = jnp.zeros((16,), jnp.int32)
        for w in range(_NS):
            row = allh_v[w, pl.ds(16 * k, 16)]
            tot16 = tot16 + row
            wlt = jnp.full((16,), w, jnp.int32) < sid16
            mine16 = mine16 + jnp.where(wlt, row, 0)
        gbase16 = carry + plsc.cumsum(tot16) - tot16
        off_v[pl.ds(16 * k, 16)] = gbase16 + mine16
        carry = carry + jnp.sum(tot16)

    for c in range(8):
        ks, vs, rank, last = runs(c)
        offs = plsc.load_gather(off_v, [ks])
        pos16 = offs + rank
        plsc.store_scatter(off_v, [ks], pos16 + 1, mask=last)
        pos_v[pl.ds(16 * c, 16)] = pos16
        tok_v[pl.ds(16 * c, 16)] = base + 16 * c + vs
        idxs_v[pl.ds(16 * c, 16)] = ks
        sr_v[pl.ds(16 * c, 16)] = plsc.load_gather(scales_v, [2 * ks])
        sf_v[pl.ds(16 * c, 16)] = plsc.load_gather(scales_v, [2 * ks + 1])
        plsc.store_scatter(post_v, [16 * c + vs], pos16)

    pltpu.async_copy(tok_v, sh_inv.at[pos_v], sem).wait()
    pltpu.async_copy(idxs_v, sh_idxs.at[pos_v], sem).wait()
    pltpu.async_copy(sr_v, sh_sr.at[pos_v], sem).wait()
    pltpu.async_copy(sf_v, sh_sf.at[pos_v], sem).wait()
    plsc.subcore_barrier()

    @pl.when(core == 0)
    def _():
        pltpu.sync_copy(sh_inv.at[pl.ds(base, 128)],
                        inv_hbm.at[pl.ds(base, 128)])
        pltpu.sync_copy(sh_idxs.at[pl.ds(base, 128)],
                        idxs_hbm.at[pl.ds(base, 128)])
        pltpu.sync_copy(sh_sr.at[pl.ds(base, 128)],
                        sr_hbm.at[pl.ds(base, 128)])
        pltpu.sync_copy(sh_sf.at[pl.ds(base, 128)],
                        sf_hbm.at[pl.ds(base, 128)])
        pltpu.sync_copy(post_v, pos_hbm.at[pl.ds(base, 128)])

    row0 = core * 1024 + sid * 64
    pltpu.sync_copy(sh_inv.at[pl.ds(row0, 64)], myinv_v)
    for j in range(4):
        pltpu.async_copy(x_hbm.at[myinv_v.at[pl.ds(16 * j, 16)]],
                         rows_v, sem).wait()
        pltpu.sync_copy(rows_v, xs_hbm.at[pl.ds(row0 + 16 * j, 16)])


def _sc_sort(idx, scales_flat, x):
    mesh = plsc.VectorSubcoreMesh(core_axis_name="c", subcore_axis_name="s")
    f = pl.kernel(
        _sc_sort_body,
        mesh=mesh,
        out_type=(
            jax.ShapeDtypeStruct((_T,), jnp.int32),
            jax.ShapeDtypeStruct((_T,), jnp.int32),
            jax.ShapeDtypeStruct((_T,), jnp.int32),
            jax.ShapeDtypeStruct((_T,), jnp.float32),
            jax.ShapeDtypeStruct((_T,), jnp.float32),
            jax.ShapeDtypeStruct((_T, _HID), jnp.float32),
        ),
        scratch_types=[
            pltpu.VMEM((128,), jnp.int32),
            pltpu.VMEM((128,), jnp.int32),
            pltpu.VMEM((128,), jnp.int32),
            pltpu.VMEM((128,), jnp.int32),
            pltpu.VMEM((128,), jnp.int32),
            pltpu.VMEM((128,), jnp.float32),
            pltpu.VMEM((128,), jnp.float32),
            pltpu.VMEM((128,), jnp.float32),
            pltpu.VMEM((_NS, _E), jnp.int32),
            pltpu.VMEM((_E,), jnp.int32),
            pltpu.VMEM((_E,), jnp.int32),
            pltpu.VMEM((32,), jnp.int32),
            pltpu.VMEM((32,), jnp.int32),
            pltpu.VMEM((64,), jnp.int32),
            pltpu.VMEM((16, _HID), jnp.float32),
            pltpu.VMEM_SHARED((_NS, _E), jnp.int32),
            pltpu.VMEM_SHARED((_T,), jnp.int32),
            pltpu.VMEM_SHARED((_T,), jnp.int32),
            pltpu.VMEM_SHARED((_T,), jnp.float32),
            pltpu.VMEM_SHARED((_T,), jnp.float32),
            pltpu.SemaphoreType.DMA,
        ],
        compiler_params=pltpu.CompilerParams(needs_layout_passes=False),
    )
    return f(idx, scales_flat, x)


def _sc_unsort_add_body(pos_hbm, ads_hbm, base_hbm, out_hbm,
                        pos_v, rows_v, base_v, sem):
    core = lax.axis_index("c")
    sid = lax.axis_index("s")
    row0 = core * 1024 + sid * 64
    pltpu.sync_copy(pos_hbm.at[pl.ds(row0, 64)], pos_v)
    for j in range(4):
        cp = pltpu.async_copy(
            ads_hbm.at[pos_v.at[pl.ds(16 * j, 16)]], rows_v, sem)
        pltpu.sync_copy(base_hbm.at[pl.ds(row0 + 16 * j, 16)], base_v)
        cp.wait()

        def addk(k, carry):
            s = pl.ds(k * 16, 16)
            for r in range(16):
                rows_v[r, s] = rows_v[r, s] + base_v[r, s]
            return carry

        lax.fori_loop(0, _HID // 16, addk, jnp.int32(0))
        pltpu.sync_copy(rows_v, out_hbm.at[pl.ds(row0 + 16 * j, 16)])


def _sc_unsort_add(pos, ad_sorted, base_out):
    mesh = plsc.VectorSubcoreMesh(core_axis_name="c", subcore_axis_name="s")
    f = pl.kernel(
        _sc_unsort_add_body,
        mesh=mesh,
        out_type=jax.ShapeDtypeStruct((_T, _HID), jnp.float32),
        scratch_types=[
            pltpu.VMEM((64,), jnp.int32),
            pltpu.VMEM((16, _HID), jnp.float32),
            pltpu.VMEM((16, _HID), jnp.float32),
            pltpu.SemaphoreType.DMA,
        ],
    )
    return f(pos, ad_sorted, base_out)


def _base_body(x_ref, bg_ref, bu_ref, bd_ref, out_ref, wg_s, wu_s, bd_s):
    c = pl.program_id(0)
    t = pl.program_id(1)

    @pl.when(t == 0)
    def _():
        wg_s[...] = bg_ref[...].astype(jnp.bfloat16)
        wu_s[...] = bu_ref[...].astype(jnp.bfloat16)
        bd_s[...] = bd_ref[...].astype(jnp.bfloat16)

    rows = pl.ds(t * _TT, _TT)
    x = x_ref[rows, :]
    dn = (((1,), (1,)), ((), ()))
    g = lax.dot_general(x, wg_s[...], dn, preferred_element_type=jnp.float32)
    u = lax.dot_general(x, wu_s[...], dn, preferred_element_type=jnp.float32)
    sig = 1.0 / (1.0 + jnp.exp(-g))
    h = (g * sig) * u
    contrib = lax.dot_general(h.astype(jnp.bfloat16), bd_s[...], dn,
                              preferred_element_type=jnp.float32)

    @pl.when(c == 0)
    def _():
        out_ref[rows, :] = contrib

    @pl.when(c > 0)
    def _():
        out_ref[rows, :] += contrib


def _tc_base(xb, base_gate_w, base_up_w, base_down_w):
    return pl.pallas_call(
        _base_body,
        grid=(_NB_BASE, _NTT),
        in_specs=[
            pl.BlockSpec((_T, _HID), lambda c, t: (0, 0)),
            pl.BlockSpec((_CBB, _HID), lambda c, t: (c, 0)),
            pl.BlockSpec((_CBB, _HID), lambda c, t: (c, 0)),
            pl.BlockSpec((_HID, _CBB), lambda c, t: (0, c)),
        ],
        out_specs=pl.BlockSpec((_T, _HID), lambda c, t: (0, 0)),
        out_shape=jax.ShapeDtypeStruct((_T, _HID), jnp.float32),
        scratch_shapes=[
            pltpu.VMEM((_CBB, _HID), jnp.bfloat16),
            pltpu.VMEM((_CBB, _HID), jnp.bfloat16),
            pltpu.VMEM((_HID, _CBB), jnp.bfloat16),
        ],
        compiler_params=pltpu.CompilerParams(
            dimension_semantics=("arbitrary", "arbitrary")),
    )(xb, base_gate_w, base_up_w, base_down_w)


def _ad_body(idx_ref, sr_ref, sf_ref, xs_ref, rg_ref, ru_ref, fg_ref, fu_ref,
             rd_ref, fd_ref, out_ref, wg_s, wu_s, wd_s):
    c = pl.program_id(0)
    t = pl.program_id(1)
    rows = pl.ds(t * _TT, _TT)

    @pl.when(t == 0)
    def _():
        @pl.when(c < _NB_AD)
        def _():
            wg_s[...] = rg_ref[...].astype(jnp.bfloat16)
            wu_s[...] = ru_ref[...].astype(jnp.bfloat16)
            wd_s[...] = rd_ref[...]

        @pl.when(c >= _NB_AD)
        def _():
            wg_s[...] = fg_ref[...].astype(jnp.bfloat16)
            wu_s[...] = fu_ref[...].astype(jnp.bfloat16)
            wd_s[...] = fd_ref[...]

    @pl.when(c == 0)
    def _():
        out_ref[rows, :] = jnp.zeros((_TT, _HID), jnp.float32)

    blk = jnp.where(c < _NB_AD, c, c - _NB_AD)
    e0 = blk * _EPB
    idxv = idx_ref[rows, :]
    emin = jnp.min(idxv)
    emax = jnp.max(idxv)

    def run(s_ref):
        x = xs_ref[rows, :]
        dn = (((1,), (1,)), ((), ()))
        g = lax.dot_general(x, wg_s[...], dn,
                            preferred_element_type=jnp.float32)
        u = lax.dot_general(x, wu_s[...], dn,
                            preferred_element_type=jnp.float32)
        sig = 1.0 / (1.0 + jnp.exp(-g))
        h = (g * sig) * u
        ecol = e0 + lax.broadcasted_iota(jnp.int32, (_TT, _CB), 1) // _NR
        h = jnp.where(ecol == idxv, h * s_ref[rows, :], 0.0)
        contrib = jnp.dot(h.astype(jnp.bfloat16), wd_s[...],
                          preferred_element_type=jnp.float32)
        out_ref[rows, :] += contrib

    @pl.when((emax >= e0) & (emin < e0 + _EPB) & (c < _NB_AD))
    def _():
        run(sr_ref)

    @pl.when((emax >= e0) & (emin < e0 + _EPB) & (c >= _NB_AD))
    def _():
        run(sf_ref)


def _tc_adapters(idxs, sr, sf, xs, rg, ru, fg, fu, rd, fd):
    nb = _NB_AD

    def rblk(c, t):
        return (jnp.clip(c, 0, nb - 1), 0)

    def fblk(c, t):
        return (jnp.clip(c - nb, 0, nb - 1), 0)

    return pl.pallas_call(
        _ad_body,
        grid=(2 * nb, _NTT),
        in_specs=[
            pl.BlockSpec((_T, 1), lambda c, t: (0, 0)),
            pl.BlockSpec((_T, 1), lambda c, t: (0, 0)),
            pl.BlockSpec((_T, 1), lambda c, t: (0, 0)),
            pl.BlockSpec((_T, _HID), lambda c, t: (0, 0)),
            pl.BlockSpec((_CB, _HID), rblk),
            pl.BlockSpec((_CB, _HID), rblk),
            pl.BlockSpec((_CB, _HID), fblk),
            pl.BlockSpec((_CB, _HID), fblk),
            pl.BlockSpec((_CB, _HID), rblk),
            pl.BlockSpec((_CB, _HID), fblk),
        ],
        out_specs=pl.BlockSpec((_T, _HID), lambda c, t: (0, 0)),
        out_shape=jax.ShapeDtypeStruct((_T, _HID), jnp.float32),
        scratch_shapes=[
            pltpu.VMEM((_CB, _HID), jnp.bfloat16),
            pltpu.VMEM((_CB, _HID), jnp.bfloat16),
            pltpu.VMEM((_CB, _HID), jnp.bfloat16),
        ],
        compiler_params=pltpu.CompilerParams(
            dimension_semantics=("arbitrary", "arbitrary")),
    )(idxs.reshape(_T, 1), sr.reshape(_T, 1), sf.reshape(_T, 1),
      xs, rg, ru, fg, fu, rd, fd)


def kernel(x, token_lora_indices, base_gate_w, base_up_w, base_down_w,
           retain_gate_stacked, retain_up_stacked, retain_down_stacked,
           forget_gate_stacked, forget_up_stacked, forget_down_stacked,
           scales):
    idx0 = jnp.maximum(token_lora_indices, 0)
    inv, pos, idxs, sr, sf, x_sorted = _sc_sort(
        idx0, scales.reshape(_E * 2), x)

    base_out = _tc_base(x.astype(jnp.bfloat16),
                        base_gate_w, base_up_w, base_down_w)

    rg = retain_gate_stacked.reshape(_E * _NR, _HID)
    ru = retain_up_stacked.reshape(_E * _NR, _HID)
    fg = forget_gate_stacked.reshape(_E * _NF, _HID)
    fu = forget_up_stacked.reshape(_E * _NF, _HID)
    rd = retain_down_stacked[:, 0].transpose(0, 2, 1).reshape(
        _E * _NR, _HID).astype(jnp.bfloat16)
    fd = forget_down_stacked[:, 0].transpose(0, 2, 1).reshape(
        _E * _NF, _HID).astype(jnp.bfloat16)

    ad_sorted = _tc_adapters(idxs, sr, sf, x_sorted.astype(jnp.bfloat16),
                             rg, ru, fg, fu, rd, fd)

    return _sc_unsort_add(pos, ad_sorted, base_out)
